# baseline (speedup 1.0000x reference)
import functools
import jax
import jax.numpy as jnp
from jax import lax
from jax.experimental import pallas as pl
from jax.experimental.pallas import tpu as pltpu
from jax.experimental.pallas import tpu_sc as plsc

M, B, D, C = 8, 16384, 128, 10
CP = 16

NC, NS, L = 2, 16, 16
NW = NC * NS
S = B // NW
RPT = S * C // 128


def _tc_votes_body(x_ref, w_ref, b_ref, votes_ref, w4_s):
    bs = x_ref.shape[0]

    @pl.when(pl.program_id(0) == 0)
    def _pack():
        w4_s[:] = jnp.concatenate([w_ref[m].T for m in range(M)], axis=0)

    xT = x_ref[:].T
    logitsT = jnp.dot(w4_s[:], xT, preferred_element_type=jnp.float32)
    l3 = logitsT.reshape(M, CP, bs)[:, :C, :]
    l3 = l3 + b_ref[:][:, :, None]
    mx = jnp.max(l3, axis=1, keepdims=True)
    iota = lax.broadcasted_iota(jnp.int32, (M, C, bs), 1)
    cand = jnp.where(l3 >= mx, iota, C)
    votes_ref[:] = jnp.min(cand, axis=1)


def _sc_hist_body(votes_hbm, out_hbm, votes_v, counts_v):
    wid = lax.axis_index("s") * NC + lax.axis_index("c")
    base = wid * S
    pltpu.sync_copy(votes_hbm.at[:, pl.ds(base, S)], votes_v)

    ones = jnp.ones((L,), jnp.float32)
    zerosf = jnp.zeros((L,), jnp.float32)
    lane = lax.iota(jnp.int32, L)

    def hist_body(i, carry):
        sbase = i * L
        samp = lane + sbase
        for c in range(C):
            plsc.store_scatter(counts_v, [samp, jnp.full((L,), c, jnp.int32)],
                               zerosf)
        for m in range(M):
            v = votes_v[m, pl.ds(sbase, L)]
            plsc.addupdate_scatter(counts_v, [samp, v], ones)
        return carry

    lax.fori_loop(0, S // L, hist_body, 0)
    pltpu.sync_copy(counts_v, out_hbm.at[pl.ds(base, S)])


def kernel(x, W, b):
    Wp = jnp.pad(W, ((0, 0), (0, 0), (0, CP - C)))
    bs = 4096
    votes = pl.pallas_call(
        _tc_votes_body,
        grid=(B // bs,),
        in_specs=[
            pl.BlockSpec((bs, D), lambda i: (i, 0)),
            pl.BlockSpec((M, D, CP), lambda i: (0, 0, 0)),
            pl.BlockSpec((M, C), lambda i: (0, 0)),
        ],
        out_specs=pl.BlockSpec((M, bs), lambda i: (0, i)),
        out_shape=jax.ShapeDtypeStruct((M, B), jnp.int32),
        scratch_shapes=[pltpu.VMEM((M * CP, D), jnp.float32)],
    )(x, Wp, b)

    mesh = plsc.VectorSubcoreMesh(core_axis_name="c", subcore_axis_name="s")
    sc_hist = functools.partial(
        pl.kernel,
        mesh=mesh,
        compiler_params=pltpu.CompilerParams(
            needs_layout_passes=False, skip_device_barrier=True,
            disable_semaphore_checks=True),
        out_type=jax.ShapeDtypeStruct((B, C), jnp.float32),
        scratch_types=[
            pltpu.VMEM((M, S), jnp.int32),
            pltpu.VMEM((S, C), jnp.float32),
        ],
    )(_sc_hist_body)
    return sc_hist(votes)


# speedup vs baseline: 1.0446x; 1.0446x over previous
import functools
import jax
import jax.numpy as jnp
from jax import lax
from jax.experimental import pallas as pl
from jax.experimental.pallas import tpu as pltpu
from jax.experimental.pallas import tpu_sc as plsc

M, B, D, C = 8, 16384, 128, 10
CP = 16

NC, NS, L = 2, 16, 16
NW = NC * NS
S = B // NW
RPT = S * C // 128


def _tc_votes_body(x_ref, w_ref, b_ref, votes_ref, w4_s):
    bs = x_ref.shape[0]

    @pl.when(pl.program_id(0) == 0)
    def _pack():
        w4_s[:] = jnp.concatenate([w_ref[m].T for m in range(M)], axis=0)

    xT = x_ref[:].T
    logitsT = jnp.dot(w4_s[:], xT, preferred_element_type=jnp.float32)
    l3 = logitsT.reshape(M, CP, bs)[:, :C, :]
    l3 = l3 + b_ref[:][:, :, None]
    mx = jnp.max(l3, axis=1, keepdims=True)
    iota = lax.broadcasted_iota(jnp.int32, (M, C, bs), 1)
    cand = jnp.where(l3 >= mx, iota, C)
    votes_ref[:] = jnp.min(cand, axis=1)


def _sc_hist_body(votes_hbm, out_hbm, votes_v, counts_v, sem):
    wid = lax.axis_index("s") * NC + lax.axis_index("c")
    base = wid * S
    pltpu.sync_copy(votes_hbm.at[:, pl.ds(base, S)], votes_v)

    ones = jnp.ones((L,), jnp.float32)
    zerosf = jnp.zeros((L,), jnp.float32)
    lane = lax.iota(jnp.int32, L)

    def hist_body(i, carry):
        sbase = i * L
        samp = lane + sbase
        for c in range(C):
            plsc.store_scatter(counts_v, [samp, jnp.full((L,), c, jnp.int32)],
                               zerosf)
        for m in range(M):
            v = votes_v[m, pl.ds(sbase, L)]
            plsc.addupdate_scatter(counts_v, [samp, v], ones)
        return carry

    NCHUNK = 4
    CH = S // NCHUNK
    copies = []
    for k in range(NCHUNK):
        lax.fori_loop(k * (CH // L), (k + 1) * (CH // L), hist_body, 0)
        copies.append(pltpu.async_copy(
            counts_v.at[pl.ds(k * CH, CH)],
            out_hbm.at[pl.ds(base + k * CH, CH)], sem))
    for cp in copies:
        cp.wait()


def kernel(x, W, b):
    Wp = jnp.pad(W, ((0, 0), (0, 0), (0, CP - C)))
    bs = 4096
    votes = pl.pallas_call(
        _tc_votes_body,
        grid=(B // bs,),
        in_specs=[
            pl.BlockSpec((bs, D), lambda i: (i, 0)),
            pl.BlockSpec((M, D, CP), lambda i: (0, 0, 0)),
            pl.BlockSpec((M, C), lambda i: (0, 0)),
        ],
        out_specs=pl.BlockSpec((M, bs), lambda i: (0, i)),
        out_shape=jax.ShapeDtypeStruct((M, B), jnp.int32),
        scratch_shapes=[pltpu.VMEM((M * CP, D), jnp.float32)],
    )(x, Wp, b)

    mesh = plsc.VectorSubcoreMesh(core_axis_name="c", subcore_axis_name="s")
    sc_hist = functools.partial(
        pl.kernel,
        mesh=mesh,
        compiler_params=pltpu.CompilerParams(
            needs_layout_passes=False, skip_device_barrier=True,
            disable_semaphore_checks=True),
        out_type=jax.ShapeDtypeStruct((B, C), jnp.float32),
        scratch_types=[
            pltpu.VMEM((M, S), jnp.int32),
            pltpu.VMEM((S, C), jnp.float32),
            pltpu.SemaphoreType.DMA,
        ],
    )(_sc_hist_body)
    return sc_hist(votes)
